# Initial kernel scaffold; baseline (speedup 1.0000x reference)
import functools

import jax
import jax.numpy as jnp
from jax import lax
from jax.experimental import pallas as pl
from jax.experimental.pallas import tpu as pltpu
from jax.experimental.pallas import tpu_sc as plsc

T = 32768
NC, NS = 2, 16
NW = NC * NS
TPW = T // NW
DPROJ = 1024
DIMS = (1024, 256, 64, 16)
LOS = (0, 20000, 100000, 500000)
HIS = (20000, 100000, 500000, 1000000)
SCALE = float(DPROJ) ** 0.5

CHUNKS = (64, 128, 128, 128)

_sc_mesh = plsc.VectorSubcoreMesh(
    core_axis_name="c", subcore_axis_name="s", num_cores=NC, num_subcores=NS
)


def _gather_body(inp_h, e0, e1, e2, e3, u0, u1, u2, u3,
                 tok_v, i0, i1, i2, i3, r0, r1, r2, r3, sem):
    wid = lax.axis_index("s") * NC + lax.axis_index("c")
    base = wid * TPW
    pltpu.sync_copy(inp_h.at[pl.ds(base, TPW)], tok_v)

    for j in range(TPW // 16):
        x = tok_v[pl.ds(j * 16, 16)]
        i0[pl.ds(j * 16, 16)] = jnp.minimum(x, 19999)
        i1[pl.ds(j * 16, 16)] = jnp.clip(x - 20000, 0, 79999)
        i2[pl.ds(j * 16, 16)] = jnp.clip(x - 100000, 0, 399999)
        i3[pl.ds(j * 16, 16)] = jnp.clip(x - 500000, 0, 499999)

    for tbl, idx, row_buf, u, ck in (
        (e0, i0, r0, u0, CHUNKS[0]),
        (e1, i1, r1, u1, CHUNKS[1]),
        (e2, i2, r2, u2, CHUNKS[2]),
        (e3, i3, r3, u3, CHUNKS[3]),
    ):
        for c in range(TPW // ck):
            pltpu.async_copy(tbl.at[idx.at[pl.ds(c * ck, ck)]], row_buf, sem).wait()
            pltpu.sync_copy(row_buf, u.at[pl.ds(base + c * ck, ck)])


_gather = pl.kernel(
    _gather_body,
    out_type=tuple(
        jax.ShapeDtypeStruct((T, d), jnp.float32) for d in DIMS
    ),
    mesh=_sc_mesh,
    scratch_types=[
        pltpu.VMEM((TPW,), jnp.int32),
        pltpu.VMEM((TPW,), jnp.int32),
        pltpu.VMEM((TPW,), jnp.int32),
        pltpu.VMEM((TPW,), jnp.int32),
        pltpu.VMEM((TPW,), jnp.int32),
        pltpu.VMEM((CHUNKS[0], DIMS[0]), jnp.float32),
        pltpu.VMEM((CHUNKS[1], DIMS[1]), jnp.float32),
        pltpu.VMEM((CHUNKS[2], DIMS[2]), jnp.float32),
        pltpu.VMEM((CHUNKS[3], DIMS[3]), jnp.float32),
        pltpu.SemaphoreType.DMA,
    ],
    name="adaptive_emb_gather",
)

BM = 256


def _mm_body(x_ref, u0_ref, u1_ref, u2_ref, u3_ref,
             w0_ref, w1_ref, w2_ref, w3_ref, o_ref):
    x = x_ref[...]
    acc = jnp.zeros((BM, DPROJ), dtype=jnp.float32)
    for u_ref, w_ref, lo, hi in (
        (u0_ref, w0_ref, LOS[0], HIS[0]),
        (u1_ref, w1_ref, LOS[1], HIS[1]),
        (u2_ref, w2_ref, LOS[2], HIS[2]),
        (u3_ref, w3_ref, LOS[3], HIS[3]),
    ):
        m = ((x >= lo) & (x < hi)).astype(jnp.float32)
        a = u_ref[...] * m
        acc = acc + lax.dot_general(
            a, w_ref[...], (((1,), (1,)), ((), ())),
            preferred_element_type=jnp.float32)
    o_ref[...] = acc * SCALE


def _matmul(inp2d, u0, u1, u2, u3, w0, w1, w2, w3):
    return pl.pallas_call(
        _mm_body,
        grid=(T // BM,),
        in_specs=[
            pl.BlockSpec((BM, 1), lambda i: (i, 0)),
            pl.BlockSpec((BM, DIMS[0]), lambda i: (i, 0)),
            pl.BlockSpec((BM, DIMS[1]), lambda i: (i, 0)),
            pl.BlockSpec((BM, DIMS[2]), lambda i: (i, 0)),
            pl.BlockSpec((BM, DIMS[3]), lambda i: (i, 0)),
            pl.BlockSpec((DPROJ, DIMS[0]), lambda i: (0, 0)),
            pl.BlockSpec((DPROJ, DIMS[1]), lambda i: (0, 0)),
            pl.BlockSpec((DPROJ, DIMS[2]), lambda i: (0, 0)),
            pl.BlockSpec((DPROJ, DIMS[3]), lambda i: (0, 0)),
        ],
        out_specs=pl.BlockSpec((BM, DPROJ), lambda i: (i, 0)),
        out_shape=jax.ShapeDtypeStruct((T, DPROJ), jnp.float32),
        name="adaptive_emb_matmul",
    )(inp2d, u0, u1, u2, u3, w0, w1, w2, w3)


def kernel(inp, emb0, emb1, emb2, emb3, proj0, proj1, proj2, proj3):
    inp_flat = inp.reshape(-1).astype(jnp.int32)
    u0, u1, u2, u3 = _gather(inp_flat, emb0, emb1, emb2, emb3)
    out = _matmul(inp_flat.reshape(T, 1), u0, u1, u2, u3,
                  proj0, proj1, proj2, proj3)
    return out.reshape(inp.shape + (DPROJ,))


# speedup vs baseline: 1.4034x; 1.4034x over previous
import functools

import jax
import jax.numpy as jnp
from jax import lax
from jax.experimental import pallas as pl
from jax.experimental.pallas import tpu as pltpu
from jax.experimental.pallas import tpu_sc as plsc

T = 32768
NC, NS = 2, 16
NW = NC * NS
TPW = T // NW
DPROJ = 1024
SCALE = float(DPROJ) ** 0.5

CK0, CK1, CK2, CK3 = 64, 64, 128, 128

_sc_mesh = plsc.VectorSubcoreMesh(
    core_axis_name="c", subcore_axis_name="s", num_cores=NC, num_subcores=NS
)


def _gather_body(inp_h, e0, e1, e2, e3, u0, u1, u2, u3,
                 tok_v, i0, i1, i2, i3, r0, r1, r2, r3, sem):
    wid = lax.axis_index("s") * NC + lax.axis_index("c")
    base = wid * TPW
    pltpu.sync_copy(inp_h.at[pl.ds(base, TPW)], tok_v)

    for j in range(TPW // 16):
        x = tok_v[pl.ds(j * 16, 16)]
        i0[pl.ds(j * 16, 16)] = jnp.minimum(x, 19999)
        i1[pl.ds(j * 16, 16)] = jnp.clip(x - 20000, 0, 79999)
        i2[pl.ds(j * 16, 16)] = jnp.clip(x - 100000, 0, 399999) >> 1
        i3[pl.ds(j * 16, 16)] = jnp.clip(x - 500000, 0, 499999) >> 3

    for tbl, idx, row_buf, u, ck in (
        (e0, i0, r0, u0, CK0),
        (e1, i1, r1, u1, CK1),
        (e2, i2, r2, u2, CK2),
        (e3, i3, r3, u3, CK3),
    ):
        for c in range(TPW // ck):
            pltpu.async_copy(tbl.at[idx.at[pl.ds(c * ck, ck)]], row_buf, sem).wait()
            pltpu.sync_copy(row_buf, u.at[pl.ds(base + c * ck, ck)])


_gather = pl.kernel(
    _gather_body,
    out_type=(
        jax.ShapeDtypeStruct((T, 1024), jnp.float32),
        jax.ShapeDtypeStruct((T, 256), jnp.float32),
        jax.ShapeDtypeStruct((T, 128), jnp.float32),
        jax.ShapeDtypeStruct((T, 128), jnp.float32),
    ),
    mesh=_sc_mesh,
    scratch_types=[
        pltpu.VMEM((TPW,), jnp.int32),
        pltpu.VMEM((TPW,), jnp.int32),
        pltpu.VMEM((TPW,), jnp.int32),
        pltpu.VMEM((TPW,), jnp.int32),
        pltpu.VMEM((TPW,), jnp.int32),
        pltpu.VMEM((CK0, 1024), jnp.float32),
        pltpu.VMEM((CK1, 256), jnp.float32),
        pltpu.VMEM((CK2, 128), jnp.float32),
        pltpu.VMEM((CK3, 128), jnp.float32),
        pltpu.SemaphoreType.DMA,
    ],
    name="adaptive_emb_gather",
)

BM = 256


def _mm_body(x_ref, u0_ref, u1_ref, u2_ref, u3_ref,
             w0_ref, w1_ref, w2_ref, w3_ref, o_ref):
    x = x_ref[...]
    f32 = jnp.float32

    def dot(a, w_ref):
        return lax.dot_general(a, w_ref[...], (((1,), (1,)), ((), ())),
                               preferred_element_type=f32)

    m0 = (x < 20000).astype(f32)
    m1 = ((x >= 20000) & (x < 100000)).astype(f32)
    m2 = ((x >= 100000) & (x < 500000)).astype(f32)
    m3 = (x >= 500000).astype(f32)

    acc = dot(u0_ref[...] * m0, w0_ref) + dot(u1_ref[...] * m1, w1_ref)

    lane = lax.broadcasted_iota(jnp.int32, (BM, 128), 1)
    v2 = jnp.clip(x - 100000, 0, 399999)
    sel2 = ((lane >> 6) == (v2 & 1)).astype(f32) * m2
    acc = acc + dot(u2_ref[...] * sel2, w2_ref)

    v3 = jnp.clip(x - 500000, 0, 499999)
    sel3 = ((lane >> 4) == (v3 & 7)).astype(f32) * m3
    acc = acc + dot(u3_ref[...] * sel3, w3_ref)

    o_ref[...] = acc * SCALE


def _matmul(inp2d, u0, u1, u2, u3, w0, w1, w22, w38):
    return pl.pallas_call(
        _mm_body,
        grid=(T // BM,),
        in_specs=[
            pl.BlockSpec((BM, 1), lambda i: (i, 0)),
            pl.BlockSpec((BM, 1024), lambda i: (i, 0)),
            pl.BlockSpec((BM, 256), lambda i: (i, 0)),
            pl.BlockSpec((BM, 128), lambda i: (i, 0)),
            pl.BlockSpec((BM, 128), lambda i: (i, 0)),
            pl.BlockSpec((DPROJ, 1024), lambda i: (0, 0)),
            pl.BlockSpec((DPROJ, 256), lambda i: (0, 0)),
            pl.BlockSpec((DPROJ, 128), lambda i: (0, 0)),
            pl.BlockSpec((DPROJ, 128), lambda i: (0, 0)),
        ],
        out_specs=pl.BlockSpec((BM, DPROJ), lambda i: (i, 0)),
        out_shape=jax.ShapeDtypeStruct((T, DPROJ), jnp.float32),
        name="adaptive_emb_matmul",
    )(inp2d, u0, u1, u2, u3, w0, w1, w22, w38)


def kernel(inp, emb0, emb1, emb2, emb3, proj0, proj1, proj2, proj3):
    inp_flat = inp.reshape(-1).astype(jnp.int32)
    e2p = emb2.reshape(200000, 128)
    e3p = emb3.reshape(62500, 128)
    u0, u1, u2, u3 = _gather(inp_flat, emb0, emb1, e2p, e3p)
    w22 = jnp.concatenate([proj2, proj2], axis=1)
    w38 = jnp.concatenate([proj3] * 8, axis=1)
    out = _matmul(inp_flat.reshape(T, 1), u0, u1, u2, u3,
                  proj0, proj1, w22, w38)
    return out.reshape(inp.shape + (DPROJ,))
